# baseline (speedup 1.0000x reference)
import functools

import jax
import jax.numpy as jnp
from jax import lax
from jax.experimental import pallas as pl
from jax.experimental.pallas import tpu as pltpu
from jax.experimental.pallas import tpu_sc as plsc

BATCH = 16384
DIM = 64
VOCAB1 = 1000001

_NC = 2
_NS = 16
_NW = _NC * _NS
_I_PER_W = 31360
_WIN = 512
_N_WIN = 62
_PHYS_MINOR = 1000064
_MAX_BASE = _PHYS_MINOR - _WIN
_LCAP = BATCH + 16
_E2_ROWS = BATCH + 16
_JBITS = 15
_JMASK = (1 << _JBITS) - 1


def _sc_gather(idx, table_t):
    mesh = plsc.VectorSubcoreMesh(core_axis_name="c", subcore_axis_name="s")

    @functools.partial(
        pl.kernel,
        mesh=mesh,
        out_type=jax.ShapeDtypeStruct((_E2_ROWS, 128), jnp.float32),
        scratch_types=[
            pltpu.VMEM((BATCH,), jnp.int32),
            pltpu.VMEM((_LCAP,), jnp.int32),
            pltpu.VMEM((_LCAP,), jnp.int32),
            pltpu.VMEM((DIM, _WIN), jnp.float32),
            pltpu.VMEM((DIM, _WIN), jnp.float32),
            pltpu.VMEM((16, 128), jnp.float32),
            pltpu.VMEM((1, 16), jnp.int32),
            pltpu.SemaphoreType.DMA,
            pltpu.SemaphoreType.DMA,
        ],
        compiler_params=pltpu.CompilerParams(needs_layout_passes=False),
    )
    def gather_kernel(idx_hbm, table_hbm, e2_hbm, idx_v, l_v, m_v,
                      win0, win1, stage, jbuf, sem_w, sem_s):
        wid = lax.axis_index("s") * _NC + lax.axis_index("c")
        i0 = wid * _I_PER_W
        i1 = jnp.minimum(i0 + _I_PER_W, VOCAB1)
        rel1 = i1 - i0
        lanes = lax.iota(jnp.int32, 16)
        dumpslots = jnp.full((16,), BATCH, jnp.int32) + lanes

        pltpu.sync_copy(idx_hbm, idx_v)

        def compact_store(dst, pv, m, cnt):
            c = plsc.all_reduce_population_count(m)

            @pl.when(c[0] > 0)
            def _():
                pos = plsc.cumsum(m.astype(jnp.int32))
                slot = jnp.where(m, cnt + pos - 1, dumpslots)
                plsc.store_scatter(dst, [slot], pv)

            return cnt + c[0]

        def filt(k, cnt):
            vi = idx_v[pl.ds(k * 16, 16)]
            m = (vi >= i0) & (vi < i1)
            pv = ((vi - i0) << _JBITS) | (lanes + k * 16)
            return compact_store(l_v, pv, m, cnt)

        cnt = lax.fori_loop(0, BATCH // 16, filt, 0)
        nch = (cnt + 15) // 16

        def band_base(w):
            return pl.multiple_of(
                jnp.minimum(i0 + w * _WIN, _MAX_BASE), 128
            )

        def start_band(w, buf):
            pltpu.make_async_copy(
                table_hbm.at[:, pl.ds(band_base(w), _WIN)], buf, sem_w
            ).start()

        start_band(0, win0)
        start_band(1, win1)

        def do_window(w, buf, nxt, sc_issued):
            base_rel = band_base(w) - i0
            s0r = w * _WIN
            s1r = s0r + _WIN
            pltpu.make_async_copy(
                table_hbm.at[:, pl.ds(base_rel + i0, _WIN)], buf, sem_w
            ).wait()

            def sel(k, mcnt):
                pv = l_v[pl.ds(k * 16, 16)]
                pr = pv >> _JBITS
                valid = (k * 16 + lanes) < cnt
                m = (pr >= s0r) & (pr < s1r) & valid
                pm = pv - (base_rel << _JBITS)
                return compact_store(m_v, pm, m, mcnt)

            mcnt = lax.fori_loop(0, nch, sel, 0)
            m_v[pl.ds(mcnt, 16)] = jnp.full((16,), BATCH, jnp.int32) + lanes

            def chunk(c, issued):
                qb = c * 16

                @pl.when(issued > 0)
                def _():
                    pltpu.make_async_copy(
                        stage, e2_hbm.at[jbuf.at[0]], sem_s
                    ).wait()

                pm = m_v[pl.ds(qb, 16)]
                vrel = jnp.clip(pm >> _JBITS, 0, _WIN - 1)
                jbuf[0, pl.ds(0, 16)] = pm & _JMASK
                for l in range(16):
                    col = jnp.broadcast_to(vrel[l], (16,))
                    for h in range(4):
                        stage[l, pl.ds(h * 16, 16)] = plsc.load_gather(
                            buf, [lanes + h * 16, col]
                        )
                pltpu.make_async_copy(
                    stage, e2_hbm.at[jbuf.at[0]], sem_s
                ).start()
                return issued + 1

            out = lax.fori_loop(0, (mcnt + 15) // 16, chunk, sc_issued)

            @pl.when(w + 2 < _N_WIN)
            def _():
                start_band(w + 2, nxt)

            return out

        def wpair(p, sc_issued):
            sc_issued = do_window(2 * p, win0, win0, sc_issued)
            return do_window(2 * p + 1, win1, win1, sc_issued)

        sc_issued = lax.fori_loop(0, _N_WIN // 2, wpair, 0)

        @pl.when(sc_issued > 0)
        def _():
            pltpu.make_async_copy(stage, e2_hbm.at[jbuf.at[0]], sem_s).wait()

    return gather_kernel(idx, table_t)


def _fnn_body(e_ref, w1_ref, b1_ref, w2_ref, b2_ref, out_ref):
    x = e_ref[...][:, :DIM]
    h = jnp.dot(x, w1_ref[...], preferred_element_type=jnp.float32)
    h = jnp.maximum(h + b1_ref[...], 0.0)
    out_ref[...] = (
        jnp.dot(h, w2_ref[...], preferred_element_type=jnp.float32)
        + b2_ref[...]
    )


def _tc_fnn(e2, W1, b1, W2, b2):
    blk = 2048
    grid = (BATCH // blk,)
    return pl.pallas_call(
        _fnn_body,
        grid=grid,
        in_specs=[
            pl.BlockSpec((blk, 128), lambda i: (i, 0)),
            pl.BlockSpec((DIM, DIM), lambda i: (0, 0)),
            pl.BlockSpec((1, DIM), lambda i: (0, 0)),
            pl.BlockSpec((DIM, DIM), lambda i: (0, 0)),
            pl.BlockSpec((1, DIM), lambda i: (0, 0)),
        ],
        out_specs=pl.BlockSpec((blk, DIM), lambda i: (i, 0)),
        out_shape=jax.ShapeDtypeStruct((BATCH, DIM), jnp.float32),
    )(e2, W1, b1, W2, b2)


def kernel(pk_idx, emb_table, W1, b1, W2, b2):
    idx = pk_idx.astype(jnp.int32)
    table_t = jnp.swapaxes(emb_table, 0, 1)
    e2 = _sc_gather(idx, table_t)
    return _tc_fnn(e2, W1, b1.reshape(1, DIM), W2, b2.reshape(1, DIM))


# speedup vs baseline: 1.0440x; 1.0440x over previous
import functools

import jax
import jax.numpy as jnp
from jax import lax
from jax.experimental import pallas as pl
from jax.experimental.pallas import tpu as pltpu
from jax.experimental.pallas import tpu_sc as plsc

BATCH = 16384
DIM = 64
VOCAB1 = 1000001

_NC = 2
_NS = 16
_NW = _NC * _NS
_I_PER_W = 31360
_WIN = 512
_N_WIN = 62
_PHYS_MINOR = 1000064
_MAX_BASE = _PHYS_MINOR - _WIN
_LCAP = BATCH + 16
_E2_ROWS = BATCH + 16
_JBITS = 15
_JMASK = (1 << _JBITS) - 1


def _sc_gather(idx, table_t):
    mesh = plsc.VectorSubcoreMesh(core_axis_name="c", subcore_axis_name="s")

    @functools.partial(
        pl.kernel,
        mesh=mesh,
        out_type=jax.ShapeDtypeStruct((_E2_ROWS, 128), jnp.float32),
        scratch_types=[
            pltpu.VMEM((BATCH,), jnp.int32),
            pltpu.VMEM((_LCAP,), jnp.int32),
            pltpu.VMEM((_LCAP,), jnp.int32),
            pltpu.VMEM((DIM, _WIN), jnp.float32),
            pltpu.VMEM((DIM, _WIN), jnp.float32),
            pltpu.VMEM((16, 128), jnp.float32),
            pltpu.VMEM((1, 16), jnp.int32),
            pltpu.SemaphoreType.DMA,
            pltpu.SemaphoreType.DMA,
        ],
        compiler_params=pltpu.CompilerParams(needs_layout_passes=False),
    )
    def gather_kernel(idx_hbm, table_hbm, e2_hbm, idx_v, l_v, m_v,
                      win0, win1, stage, jbuf, sem_w, sem_s):
        wid = lax.axis_index("s") * _NC + lax.axis_index("c")
        i0 = wid * _I_PER_W
        i1 = jnp.minimum(i0 + _I_PER_W, VOCAB1)
        rel1 = i1 - i0
        lanes = lax.iota(jnp.int32, 16)
        sent = jnp.full((16,), 2**31 - 1, jnp.int32)

        pltpu.sync_copy(idx_hbm, idx_v)

        def compact_store(dst, pv, m, cnt):
            keys = jnp.where(m, pv, sent)
            sk, _ = plsc.sort_key_val(keys, pv)
            dst[pl.ds(cnt, 16)] = sk
            c = plsc.all_reduce_population_count(m)
            return cnt + c[0]

        def filt(k, cnt):
            vi = idx_v[pl.ds(k * 16, 16)]
            m = (vi >= i0) & (vi < i1)
            pv = ((vi - i0) << _JBITS) | (lanes + k * 16)
            return compact_store(l_v, pv, m, cnt)

        cnt = lax.fori_loop(0, BATCH // 16, filt, 0)
        nch = (cnt + 15) // 16

        def band_base(w):
            return pl.multiple_of(
                jnp.minimum(i0 + w * _WIN, _MAX_BASE), 128
            )

        def start_band(w, buf):
            pltpu.make_async_copy(
                table_hbm.at[:, pl.ds(band_base(w), _WIN)], buf, sem_w
            ).start()

        start_band(0, win0)
        start_band(1, win1)

        def do_window(w, buf, nxt, sc_issued):
            base_rel = band_base(w) - i0
            s0r = w * _WIN
            s1r = s0r + _WIN
            pltpu.make_async_copy(
                table_hbm.at[:, pl.ds(base_rel + i0, _WIN)], buf, sem_w
            ).wait()

            def sel(k, mcnt):
                pv = l_v[pl.ds(k * 16, 16)]
                pr = pv >> _JBITS
                valid = (k * 16 + lanes) < cnt
                m = (pr >= s0r) & (pr < s1r) & valid
                pm = pv - (base_rel << _JBITS)
                return compact_store(m_v, pm, m, mcnt)

            mcnt = lax.fori_loop(0, nch, sel, 0)
            m_v[pl.ds(mcnt, 16)] = jnp.full((16,), BATCH, jnp.int32) + lanes

            def chunk(c, issued):
                qb = c * 16

                @pl.when(issued > 0)
                def _():
                    pltpu.make_async_copy(
                        stage, e2_hbm.at[jbuf.at[0]], sem_s
                    ).wait()

                pm = m_v[pl.ds(qb, 16)]
                vrel = jnp.clip(pm >> _JBITS, 0, _WIN - 1)
                jbuf[0, pl.ds(0, 16)] = pm & _JMASK
                for l in range(16):
                    col = jnp.broadcast_to(vrel[l], (16,))
                    for h in range(4):
                        stage[l, pl.ds(h * 16, 16)] = plsc.load_gather(
                            buf, [lanes + h * 16, col]
                        )
                pltpu.make_async_copy(
                    stage, e2_hbm.at[jbuf.at[0]], sem_s
                ).start()
                return issued + 1

            out = lax.fori_loop(0, (mcnt + 15) // 16, chunk, sc_issued)

            @pl.when(w + 2 < _N_WIN)
            def _():
                start_band(w + 2, nxt)

            return out

        def wpair(p, sc_issued):
            sc_issued = do_window(2 * p, win0, win0, sc_issued)
            return do_window(2 * p + 1, win1, win1, sc_issued)

        sc_issued = lax.fori_loop(0, _N_WIN // 2, wpair, 0)

        @pl.when(sc_issued > 0)
        def _():
            pltpu.make_async_copy(stage, e2_hbm.at[jbuf.at[0]], sem_s).wait()

    return gather_kernel(idx, table_t)


def _fnn_body(e_ref, w1_ref, b1_ref, w2_ref, b2_ref, out_ref):
    x = e_ref[...][:, :DIM]
    h = jnp.dot(x, w1_ref[...], preferred_element_type=jnp.float32)
    h = jnp.maximum(h + b1_ref[...], 0.0)
    out_ref[...] = (
        jnp.dot(h, w2_ref[...], preferred_element_type=jnp.float32)
        + b2_ref[...]
    )


def _tc_fnn(e2, W1, b1, W2, b2):
    blk = 2048
    grid = (BATCH // blk,)
    return pl.pallas_call(
        _fnn_body,
        grid=grid,
        in_specs=[
            pl.BlockSpec((blk, 128), lambda i: (i, 0)),
            pl.BlockSpec((DIM, DIM), lambda i: (0, 0)),
            pl.BlockSpec((1, DIM), lambda i: (0, 0)),
            pl.BlockSpec((DIM, DIM), lambda i: (0, 0)),
            pl.BlockSpec((1, DIM), lambda i: (0, 0)),
        ],
        out_specs=pl.BlockSpec((blk, DIM), lambda i: (i, 0)),
        out_shape=jax.ShapeDtypeStruct((BATCH, DIM), jnp.float32),
    )(e2, W1, b1, W2, b2)


def kernel(pk_idx, emb_table, W1, b1, W2, b2):
    idx = pk_idx.astype(jnp.int32)
    table_t = jnp.swapaxes(emb_table, 0, 1)
    e2 = _sc_gather(idx, table_t)
    return _tc_fnn(e2, W1, b1.reshape(1, DIM), W2, b2.reshape(1, DIM))
